# baseline (speedup 1.0000x reference)
import jax
import jax.numpy as jnp
from jax import lax
from jax.experimental import pallas as pl
from jax.experimental.pallas import tpu as pltpu
from jax.experimental.pallas import tpu_sc as plsc

D = 14336
NC, NS = 2, 16
NW = NC * NS
B = 512
BPW = B // NW
CH = 2
NB = 2
NCHUNK = BPW // CH


def _body(idx_hbm, table_hbm, out_hbm, idx_v, buf0, buf1, g0, g1, w0, w1):
    wid = lax.axis_index("s") * NC + lax.axis_index("c")
    base = wid * BPW
    pltpu.sync_copy(idx_hbm.at[wid], idx_v)
    bufs = (buf0, buf1)
    gsems = (g0, g1)
    wsems = (w0, w1)

    def gather(j, b):
        return pltpu.make_async_copy(
            table_hbm.at[idx_v.at[j]], bufs[b], gsems[b])

    def write(j, b):
        return pltpu.make_async_copy(
            bufs[b], out_hbm.at[pl.ds(base + j * CH, CH)], wsems[b])

    for b in range(NB):
        gather(b, b).start()
    def step(t, carry):
        for b in range(NB):
            j = t * NB + b
            gather(j, b).wait()
            write(j, b).start()
            write(j, b).wait()
            gather(j + NB, b).start()
        return carry
    lax.fori_loop(0, NCHUNK // NB - 1, step, 0)
    for b in range(NB):
        j = NCHUNK - NB + b
        gather(j, b).wait()
        write(j, b).start()
    for b in range(NB):
        write(NCHUNK - NB + b, b).wait()


_gather_call = pl.kernel(
    _body,
    out_type=jax.ShapeDtypeStruct((B, D), jnp.float32),
    mesh=plsc.VectorSubcoreMesh(core_axis_name="c", subcore_axis_name="s"),
    scratch_types=(
        [pltpu.VMEM((NCHUNK, CH), jnp.int32)]
        + [pltpu.VMEM((CH, D), jnp.float32)] * NB
        + [pltpu.SemaphoreType.DMA] * (2 * NB)
    ),
)


def kernel(prefix, embedding_table):
    bsz, seq = prefix.shape
    idx = prefix.astype(jnp.int32).reshape(NW, NCHUNK, CH)
    out = _gather_call(idx, embedding_table)
    return out.reshape(bsz, seq, D)
